# baseline (speedup 1.0000x reference)
import functools

import jax
import jax.numpy as jnp
from jax import lax
from jax.experimental import pallas as pl
from jax.experimental.pallas import tpu as pltpu
from jax.experimental.pallas import tpu_sc as plsc

NUM_CORES = 2
NUM_SUBCORES = 16
NUM_WORKERS = NUM_CORES * NUM_SUBCORES


@functools.cache
def _make_gather(n_i: int, n_j: int, dim: int, n_e: int):
    nib = n_i // 128
    units = n_j * nib
    upw = units // NUM_WORKERS
    assert upw * NUM_WORKERS == units and upw % 2 == 0 and upw >= 4
    nt = dim // 8
    mesh = plsc.VectorSubcoreMesh(core_axis_name="c", subcore_axis_name="s")

    @functools.partial(
        pl.kernel,
        mesh=mesh,
        out_type=jax.ShapeDtypeStruct((n_j, nt, nib, 8, 128), jnp.float32),
        scratch_types=[
            pltpu.VMEM((128,), jnp.int32),
            pltpu.VMEM((128,), jnp.int32),
            pltpu.VMEM((nt, 8, 128), jnp.float32),
            pltpu.VMEM((nt, 8, 128), jnp.float32),
            pltpu.SemaphoreType.DMA,
            pltpu.SemaphoreType.DMA,
            pltpu.SemaphoreType.DMA,
            pltpu.SemaphoreType.DMA,
            pltpu.SemaphoreType.DMA,
            pltpu.SemaphoreType.DMA,
        ],
        compiler_params=pltpu.CompilerParams(use_tc_tiling_on_sc=False),
    )
    def gather_kernel(
        idx_hbm, table_hbm, out_hbm,
        idx_a, idx_b, tile_a, tile_b,
        sia, sib, sga, sgb, soa, sob,
    ):
        wid = lax.axis_index("s") * NUM_CORES + lax.axis_index("c")
        u0 = wid * upw

        idx_v = (idx_a, idx_b)
        tile_v = (tile_a, tile_b)
        si = (sia, sib)
        sg = (sga, sgb)
        so = (soa, sob)

        def fire_idx(u, p):
            pltpu.async_copy(
                idx_hbm.at[pl.ds(u * 128, 128)], idx_v[p], si[p]
            )

        def wait_idx(p):
            pltpu.make_async_copy(
                idx_hbm.at[pl.ds(0, 128)], idx_v[p], si[p]
            ).wait()

        def fire_gathers(p):
            for d in range(dim):
                pltpu.async_copy(
                    table_hbm.at[pl.ds(d * n_e, n_e)].at[idx_v[p]],
                    tile_v[p].at[d // 8, d % 8],
                    sg[p],
                )

        def wait_gathers(p):
            for _ in range(dim):
                pltpu.make_async_copy(
                    table_hbm.at[pl.ds(0, n_e)].at[idx_v[p]],
                    tile_v[p].at[0, 0],
                    sg[p],
                ).wait()

        def out_slice(u):
            j = u // nib
            ib = u % nib
            return out_hbm.at[j, :, ib, :, :]

        def fire_out(u, p):
            pltpu.async_copy(tile_v[p], out_slice(u), so[p])

        def wait_out(u, p):
            pltpu.make_async_copy(tile_v[p], out_slice(u), so[p]).wait()

        def body(u, p, q, prefetch=True, wait_prev_out=True):
            wait_idx(p)
            if prefetch:
                fire_idx(u + 1, q)
            if wait_prev_out:
                wait_out(u - 2, p)
            fire_gathers(p)
            wait_gathers(p)
            fire_out(u, p)

        fire_idx(u0, 0)
        body(u0, 0, 1, wait_prev_out=False)
        body(u0 + 1, 1, 0, wait_prev_out=False)

        def loop_body(ii, carry):
            u = u0 + 2 * ii
            body(u, 0, 1)
            body(u + 1, 1, 0)
            return carry

        lax.fori_loop(1, upw // 2 - 1, loop_body, 0)

        body(u0 + upw - 2, 0, 1)
        body(u0 + upw - 1, 1, 0, prefetch=False)

        wait_out(u0 + upw - 2, 0)
        wait_out(u0 + upw - 1, 1)

    return gather_kernel


def kernel(indices, weight):
    n_i, n_j = indices.shape
    n_e, dim = weight.shape
    idx = indices.T.reshape(-1).astype(jnp.int32)
    table_t = weight.T.reshape(-1)
    o5 = _make_gather(n_i, n_j, dim, n_e)(idx, table_t)
    return o5.transpose((2, 4, 0, 1, 3)).reshape(n_i, n_j, dim)


# speedup vs baseline: 2.1822x; 2.1822x over previous
import functools

import jax
import jax.numpy as jnp
from jax import lax
from jax.experimental import pallas as pl
from jax.experimental.pallas import tpu as pltpu
from jax.experimental.pallas import tpu_sc as plsc

BLOCK_ROWS = 1280
NUM_CORES = 2
NUM_SUBCORES = 16
NUM_WORKERS = NUM_CORES * NUM_SUBCORES


@functools.cache
def _make_gather(num_rows_total: int, dim: int):
    rpw = num_rows_total // NUM_WORKERS
    assert rpw * NUM_WORKERS == num_rows_total
    blocks = rpw // BLOCK_ROWS
    assert blocks * BLOCK_ROWS == rpw and blocks % 2 == 0 and blocks >= 4
    mesh = plsc.VectorSubcoreMesh(core_axis_name="c", subcore_axis_name="s")

    @functools.partial(
        pl.kernel,
        mesh=mesh,
        out_type=jax.ShapeDtypeStruct((num_rows_total, dim), jnp.float32),
        scratch_types=[
            pltpu.VMEM((BLOCK_ROWS,), jnp.int32),
            pltpu.VMEM((BLOCK_ROWS,), jnp.int32),
            pltpu.VMEM((BLOCK_ROWS, dim), jnp.float32),
            pltpu.VMEM((BLOCK_ROWS, dim), jnp.float32),
            pltpu.SemaphoreType.DMA,
            pltpu.SemaphoreType.DMA,
            pltpu.SemaphoreType.DMA,
            pltpu.SemaphoreType.DMA,
            pltpu.SemaphoreType.DMA,
            pltpu.SemaphoreType.DMA,
        ],
        compiler_params=pltpu.CompilerParams(use_tc_tiling_on_sc=False),
    )
    def gather_kernel(
        idx_hbm, table_hbm, out_hbm,
        idx_a, idx_b, rows_a, rows_b,
        sia, sib, sga, sgb, soa, sob,
    ):
        wid = lax.axis_index("s") * NUM_CORES + lax.axis_index("c")
        r0 = wid * rpw

        idx_v = (idx_a, idx_b)
        rows_v = (rows_a, rows_b)
        si = (sia, sib)
        sg = (sga, sgb)
        so = (soa, sob)

        def fire_idx(b, p):
            pltpu.async_copy(
                idx_hbm.at[pl.ds(r0 + b * BLOCK_ROWS, BLOCK_ROWS)],
                idx_v[p],
                si[p],
            )

        def wait_idx(p):
            pltpu.make_async_copy(
                idx_hbm.at[pl.ds(0, BLOCK_ROWS)], idx_v[p], si[p]
            ).wait()

        def fire_gather(p):
            pltpu.async_copy(table_hbm.at[idx_v[p]], rows_v[p], sg[p])

        def wait_gather(p):
            pltpu.make_async_copy(
                table_hbm.at[idx_v[p]], rows_v[p], sg[p]
            ).wait()

        def out_slice(b):
            return out_hbm.at[pl.ds(r0 + b * BLOCK_ROWS, BLOCK_ROWS)]

        def fire_out(b, p):
            pltpu.async_copy(rows_v[p], out_slice(b), so[p])

        def wait_out(b, p):
            pltpu.make_async_copy(rows_v[p], out_slice(b), so[p]).wait()

        def body(b, p, q, fire_next=True, wait_prev_out=True,
                 has_next_idx=True):
            if fire_next:
                wait_idx(q)
                if wait_prev_out:
                    wait_out(b - 1, q)
                fire_gather(q)
            wait_gather(p)
            if has_next_idx:
                fire_idx(b + 2, p)
            fire_out(b, p)

        fire_idx(0, 0)
        fire_idx(1, 1)
        wait_idx(0)
        fire_gather(0)

        body(0, 0, 1, wait_prev_out=False)
        body(1, 1, 0)

        def loop_body(ii, carry):
            body(2 * ii, 0, 1)
            body(2 * ii + 1, 1, 0)
            return carry

        lax.fori_loop(1, blocks // 2 - 1, loop_body, 0)

        body(blocks - 2, 0, 1, has_next_idx=False)
        body(blocks - 1, 1, 0, fire_next=False, has_next_idx=False)

        wait_out(blocks - 2, 0)
        wait_out(blocks - 1, 1)

    return gather_kernel


def kernel(indices, weight):
    n_i, n_j = indices.shape
    n_e, dim = weight.shape
    idx = indices.reshape(-1).astype(jnp.int32)
    out = _make_gather(idx.shape[0], dim)(idx, weight)
    return out.reshape(n_i, n_j, dim)
